# Initial kernel scaffold; baseline (speedup 1.0000x reference)
import jax
import jax.numpy as jnp
from jax.experimental import pallas as pl


def kernel(input, weight):
    raise NotImplementedError("write your pallas kernel here")


# speedup vs baseline: 1.1623x; 1.1623x over previous
import functools

import jax
import jax.numpy as jnp
from jax import lax
from jax.experimental import pallas as pl
from jax.experimental.pallas import tpu as pltpu
from jax.experimental.pallas import tpu_sc as plsc

D = 128
CHUNK = 128
NC = 2
NS = 16
NW = NC * NS


def _make_gather(num_rows):
    assert num_rows % (NW * CHUNK) == 0
    chunks_per_w = num_rows // (NW * CHUNK)

    mesh = plsc.VectorSubcoreMesh(core_axis_name="c", subcore_axis_name="s")

    @functools.partial(
        pl.kernel,
        mesh=mesh,
        out_type=jax.ShapeDtypeStruct((num_rows, D), jnp.float32),
        scratch_types=[
            pltpu.VMEM((chunks_per_w, CHUNK), jnp.int32),
            pltpu.VMEM((CHUNK, D), jnp.float32),
            pltpu.SemaphoreType.DMA,
        ],
    )
    def gather_kernel(idx_hbm, table_hbm, out_hbm, idx_v, rows_v, sem):
        wid = lax.axis_index("s") * NC + lax.axis_index("c")
        pltpu.sync_copy(idx_hbm.at[wid], idx_v)

        def body(j, carry):
            pltpu.async_copy(table_hbm.at[idx_v.at[j]], rows_v, sem).wait()
            row_base = (wid * chunks_per_w + j) * CHUNK
            pltpu.sync_copy(rows_v, out_hbm.at[pl.ds(row_base, CHUNK)])
            return carry

        lax.fori_loop(0, chunks_per_w, body, 0)

    return gather_kernel


def kernel(input, weight):
    num_rows = input.size
    flat_idx = input.reshape(NW, num_rows // (NW * CHUNK), CHUNK).astype(jnp.int32)
    out = _make_gather(num_rows)(flat_idx, weight)
    return out.reshape(tuple(input.shape) + (weight.shape[1],))


# speedup vs baseline: 1.2111x; 1.0419x over previous
import functools

import jax
import jax.numpy as jnp
from jax import lax
from jax.experimental import pallas as pl
from jax.experimental.pallas import tpu as pltpu
from jax.experimental.pallas import tpu_sc as plsc

D = 128
CHUNK = 128
NC = 2
NS = 16
NW = NC * NS


def _make_gather(num_rows):
    assert num_rows % (NW * CHUNK) == 0
    chunks_per_w = num_rows // (NW * CHUNK)

    mesh = plsc.VectorSubcoreMesh(core_axis_name="c", subcore_axis_name="s")

    assert chunks_per_w % 2 == 0
    half = chunks_per_w // 2

    @functools.partial(
        pl.kernel,
        mesh=mesh,
        out_type=jax.ShapeDtypeStruct((num_rows, D), jnp.float32),
        scratch_types=[
            pltpu.VMEM((chunks_per_w, CHUNK), jnp.int32),
            pltpu.VMEM((CHUNK, D), jnp.float32),
            pltpu.VMEM((CHUNK, D), jnp.float32),
            pltpu.SemaphoreType.DMA,
            pltpu.SemaphoreType.DMA,
            pltpu.SemaphoreType.DMA,
            pltpu.SemaphoreType.DMA,
        ],
    )
    def gather_kernel(idx_hbm, table_hbm, out_hbm, idx_v,
                      buf0, buf1, g0, g1, w0, w1):
        wid = lax.axis_index("s") * NC + lax.axis_index("c")
        pltpu.sync_copy(idx_hbm.at[wid], idx_v)
        out_base = wid * chunks_per_w

        def gather_start(c, buf, sem):
            pltpu.async_copy(table_hbm.at[idx_v.at[c]], buf, sem)

        def gather_wait(buf, sem):
            pltpu.make_async_copy(table_hbm.at[idx_v.at[0]], buf, sem).wait()

        def write_start(c, buf, sem):
            pltpu.async_copy(buf, out_hbm.at[pl.ds((out_base + c) * CHUNK, CHUNK)], sem)

        def write_wait(buf, sem):
            pltpu.make_async_copy(buf, out_hbm.at[pl.ds(out_base * CHUNK, CHUNK)], sem).wait()

        gather_start(0, buf0, g0)

        def body(i, carry):
            c0 = 2 * i
            gather_wait(buf0, g0)
            write_start(c0, buf0, w0)
            pl.when(i > 0)(lambda: write_wait(buf1, w1))
            gather_start(c0 + 1, buf1, g1)
            gather_wait(buf1, g1)
            write_start(c0 + 1, buf1, w1)

            def refill():
                write_wait(buf0, w0)
                gather_start(c0 + 2, buf0, g0)
            pl.when(i < half - 1)(refill)
            return carry

        lax.fori_loop(0, half, body, 0)
        write_wait(buf0, w0)
        write_wait(buf1, w1)

    return gather_kernel


def kernel(input, weight):
    num_rows = input.size
    flat_idx = input.reshape(NW, num_rows // (NW * CHUNK), CHUNK).astype(jnp.int32)
    out = _make_gather(num_rows)(flat_idx, weight)
    return out.reshape(tuple(input.shape) + (weight.shape[1],))


# speedup vs baseline: 1.8002x; 1.4865x over previous
import functools

import jax
import jax.numpy as jnp
from jax import lax
from jax.experimental import pallas as pl
from jax.experimental.pallas import tpu as pltpu
from jax.experimental.pallas import tpu_sc as plsc

D = 128
NC = 2
NS = 16
NW = NC * NS
KI = 4


def _make_gather(batch, seq):
    assert batch % (NW * KI) == 0
    items_per_w = batch // NW
    chunks_per_w = items_per_w // KI
    assert chunks_per_w % 2 == 0
    half = chunks_per_w // 2

    mesh = plsc.VectorSubcoreMesh(core_axis_name="c", subcore_axis_name="s")

    @functools.partial(
        pl.kernel,
        mesh=mesh,
        out_type=jax.ShapeDtypeStruct((batch, seq, D), jnp.float32),
        scratch_types=[
            pltpu.VMEM((chunks_per_w, KI * seq), jnp.int32),
            pltpu.VMEM((KI * seq, D), jnp.float32),
            pltpu.VMEM((KI * seq, D), jnp.float32),
            pltpu.SemaphoreType.DMA,
            pltpu.SemaphoreType.DMA,
            pltpu.SemaphoreType.DMA,
            pltpu.SemaphoreType.DMA,
        ],
    )
    def gather_kernel(idx_hbm, table_hbm, out_hbm, idx_v,
                      buf0, buf1, g0, g1, w0, w1):
        wid = lax.axis_index("s") * NC + lax.axis_index("c")
        item_base = wid * items_per_w
        pltpu.sync_copy(idx_hbm.at[wid], idx_v)

        def gather_start(c, buf, sem):
            pltpu.async_copy(table_hbm.at[idx_v.at[c]], buf, sem)

        def gather_wait(buf, sem):
            pltpu.make_async_copy(
                table_hbm.at[idx_v.at[0]], buf, sem).wait()

        def write_start(c, buf, sem):
            pltpu.async_copy(
                buf.reshape(KI, seq, D),
                out_hbm.at[pl.ds(item_base + c * KI, KI)], sem)

        def write_wait(buf, sem):
            pltpu.make_async_copy(
                buf.reshape(KI, seq, D),
                out_hbm.at[pl.ds(item_base, KI)], sem).wait()

        gather_start(0, buf0, g0)

        def body(i, carry):
            c0 = 2 * i
            gather_wait(buf0, g0)
            write_start(c0, buf0, w0)
            pl.when(i > 0)(lambda: write_wait(buf1, w1))
            gather_start(c0 + 1, buf1, g1)
            gather_wait(buf1, g1)
            write_start(c0 + 1, buf1, w1)

            def refill():
                write_wait(buf0, w0)
                gather_start(c0 + 2, buf0, g0)
            pl.when(i < half - 1)(refill)
            return carry

        lax.fori_loop(0, half, body, 0)
        write_wait(buf0, w0)
        write_wait(buf1, w1)

    return gather_kernel


def kernel(input, weight):
    batch, seq = input.shape
    items_per_w = batch // NW
    idx = input.astype(jnp.int32).reshape(NW, items_per_w // KI, KI * seq)
    out = _make_gather(batch, seq)(idx, weight)
    return out


# speedup vs baseline: 2.0692x; 1.1494x over previous
import functools

import jax
import jax.numpy as jnp
from jax import lax
from jax.experimental import pallas as pl
from jax.experimental.pallas import tpu as pltpu
from jax.experimental.pallas import tpu_sc as plsc

D = 128
NC = 2
NS = 16
NW = NC * NS
KI = 4


def _make_gather(batch, seq):
    assert batch % (NW * KI) == 0
    items_per_w = batch // NW
    chunks_per_w = items_per_w // KI
    assert chunks_per_w % 4 == 0
    groups = chunks_per_w // 4

    mesh = plsc.VectorSubcoreMesh(core_axis_name="c", subcore_axis_name="s")

    @functools.partial(
        pl.kernel,
        mesh=mesh,
        out_type=jax.ShapeDtypeStruct((batch, seq, D), jnp.float32),
        scratch_types=[
            pltpu.VMEM((chunks_per_w, KI * seq), jnp.int32),
            pltpu.VMEM((KI * seq, D), jnp.float32),
            pltpu.VMEM((KI * seq, D), jnp.float32),
            pltpu.VMEM((KI * seq, D), jnp.float32),
            pltpu.VMEM((KI * seq, D), jnp.float32),
            pltpu.SemaphoreType.DMA,
            pltpu.SemaphoreType.DMA,
            pltpu.SemaphoreType.DMA,
            pltpu.SemaphoreType.DMA,
            pltpu.SemaphoreType.DMA,
            pltpu.SemaphoreType.DMA,
            pltpu.SemaphoreType.DMA,
            pltpu.SemaphoreType.DMA,
        ],
    )
    def gather_kernel(idx_hbm, table_hbm, out_hbm, idx_v,
                      b0, b1, b2, b3, g0, g1, g2, g3, w0, w1, w2, w3):
        wid = lax.axis_index("s") * NC + lax.axis_index("c")
        item_base = wid * items_per_w
        pltpu.sync_copy(idx_hbm.at[wid], idx_v)

        def gather_start(c, buf, sem):
            pltpu.async_copy(table_hbm.at[idx_v.at[c]], buf, sem)

        def gather_wait(buf, sem):
            pltpu.make_async_copy(
                table_hbm.at[idx_v.at[0]], buf, sem).wait()

        def write_start(c, buf, sem):
            pltpu.async_copy(
                buf.reshape(KI, seq, D),
                out_hbm.at[pl.ds(item_base + c * KI, KI)], sem)

        def write_wait(buf, sem):
            pltpu.make_async_copy(
                buf.reshape(KI, seq, D),
                out_hbm.at[pl.ds(item_base, KI)], sem).wait()

        bufs = [b0, b1, b2, b3]
        gs = [g0, g1, g2, g3]
        ws = [w0, w1, w2, w3]

        gather_start(0, bufs[0], gs[0])
        gather_start(1, bufs[1], gs[1])
        gather_start(2, bufs[2], gs[2])

        def body(i, carry):
            for k in range(4):
                c = 4 * i + k
                gather_wait(bufs[k], gs[k])
                write_start(c, bufs[k], ws[k])
                kk = (k + 3) % 4

                def refill(c=c, kk=kk):
                    write_wait(bufs[kk], ws[kk])
                    gather_start(c + 3, bufs[kk], gs[kk])

                if k == 0:
                    pl.when(i > 0)(refill)
                    pl.when(i == 0)(
                        lambda: gather_start(3, bufs[3], gs[3]))
                else:
                    pl.when(c + 3 < chunks_per_w)(refill)
            return carry

        lax.fori_loop(0, groups, body, 0)
        for k in range(4):
            write_wait(bufs[k], ws[k])

    return gather_kernel


def kernel(input, weight):
    batch, seq = input.shape
    items_per_w = batch // NW
    idx = input.astype(jnp.int32).reshape(NW, items_per_w // KI, KI * seq)
    out = _make_gather(batch, seq)(idx, weight)
    return out


# speedup vs baseline: 3.7997x; 1.8363x over previous
import functools

import jax
import jax.numpy as jnp
from jax import lax
from jax.experimental import pallas as pl
from jax.experimental.pallas import tpu as pltpu
from jax.experimental.pallas import tpu_sc as plsc

D = 128
NC = 2
NS = 16
NW = NC * NS


def _make_gather(batch, seq):
    assert batch % NW == 0
    bpw = batch // NW
    nchunks = seq
    full_groups = nchunks // 4
    rem = nchunks % 4

    mesh = plsc.VectorSubcoreMesh(core_axis_name="c", subcore_axis_name="s")

    @functools.partial(
        pl.kernel,
        mesh=mesh,
        out_type=jax.ShapeDtypeStruct((seq, batch, D), jnp.float32),
        scratch_types=[
            pltpu.VMEM((seq, bpw), jnp.int32),
            pltpu.VMEM((bpw, D), jnp.float32),
            pltpu.VMEM((bpw, D), jnp.float32),
            pltpu.VMEM((bpw, D), jnp.float32),
            pltpu.VMEM((bpw, D), jnp.float32),
            pltpu.SemaphoreType.DMA,
            pltpu.SemaphoreType.DMA,
            pltpu.SemaphoreType.DMA,
            pltpu.SemaphoreType.DMA,
            pltpu.SemaphoreType.DMA,
            pltpu.SemaphoreType.DMA,
            pltpu.SemaphoreType.DMA,
            pltpu.SemaphoreType.DMA,
        ],
    )
    def gather_kernel(idx_hbm, table_hbm, out_hbm, idx_v,
                      b0, b1, b2, b3, g0, g1, g2, g3, w0, w1, w2, w3):
        wid = lax.axis_index("s") * NC + lax.axis_index("c")
        col_base = wid * bpw
        pltpu.sync_copy(idx_hbm.at[:, pl.ds(col_base, bpw)], idx_v)

        def gather_start(c, buf, sem):
            pltpu.async_copy(table_hbm.at[idx_v.at[c]], buf, sem)

        def gather_wait(buf, sem):
            pltpu.make_async_copy(
                table_hbm.at[idx_v.at[0]], buf, sem).wait()

        def write_start(c, buf, sem):
            pltpu.async_copy(buf, out_hbm.at[c, pl.ds(col_base, bpw)], sem)

        def write_wait(buf, sem):
            pltpu.make_async_copy(
                buf, out_hbm.at[0, pl.ds(col_base, bpw)], sem).wait()

        bufs = [b0, b1, b2, b3]
        gs = [g0, g1, g2, g3]
        ws = [w0, w1, w2, w3]

        gather_start(0, bufs[0], gs[0])
        gather_start(1, bufs[1], gs[1])
        gather_start(2, bufs[2], gs[2])

        def body(i, carry):
            for k in range(4):
                c = 4 * i + k
                gather_wait(bufs[k], gs[k])
                write_start(c, bufs[k], ws[k])
                kk = (k + 3) % 4

                def refill(c=c, kk=kk):
                    write_wait(bufs[kk], ws[kk])
                    gather_start(c + 3, bufs[kk], gs[kk])

                if k == 0:
                    pl.when(i > 0)(refill)
                    pl.when(i == 0)(
                        lambda: gather_start(3, bufs[3], gs[3]))
                else:
                    pl.when(c + 3 < nchunks)(refill)
            return carry

        lax.fori_loop(0, full_groups, body, 0)
        for k in range(rem):
            c = 4 * full_groups + k
            gather_wait(bufs[k], gs[k])
            write_start(c, bufs[k], ws[k])
        for k in range(4):
            write_wait(bufs[k], ws[k])

    return gather_kernel


def kernel(input, weight):
    batch, seq = input.shape
    idx_t = input.astype(jnp.int32).T
    out_t = _make_gather(batch, seq)(idx_t, weight)
    return jnp.transpose(out_t, (1, 0, 2))


# speedup vs baseline: 3.8834x; 1.0220x over previous
import functools

import jax
import jax.numpy as jnp
from jax import lax
from jax.experimental import pallas as pl
from jax.experimental.pallas import tpu as pltpu
from jax.experimental.pallas import tpu_sc as plsc

D = 128
NC = 2
NS = 16
NW = NC * NS
NB = 6


def _make_gather(batch, seq):
    assert batch % NW == 0
    bpw = batch // NW
    nchunks = seq
    full_groups = nchunks // NB
    rem = nchunks % NB

    mesh = plsc.VectorSubcoreMesh(core_axis_name="c", subcore_axis_name="s")

    @functools.partial(
        pl.kernel,
        mesh=mesh,
        out_type=jax.ShapeDtypeStruct((seq, batch, D), jnp.float32),
        scratch_types=(
            [pltpu.VMEM((seq, bpw), jnp.int32)]
            + [pltpu.VMEM((bpw, D), jnp.float32)] * NB
            + [pltpu.SemaphoreType.DMA] * (2 * NB)
        ),
    )
    def gather_kernel(idx_hbm, table_hbm, out_hbm, idx_v, *bufs_sems):
        bufs = bufs_sems[:NB]
        gs = bufs_sems[NB:2 * NB]
        ws = bufs_sems[2 * NB:]
        wid = lax.axis_index("s") * NC + lax.axis_index("c")
        col_base = wid * bpw
        pltpu.sync_copy(idx_hbm.at[:, pl.ds(col_base, bpw)], idx_v)

        def gather_start(c, buf, sem):
            pltpu.async_copy(table_hbm.at[idx_v.at[c]], buf, sem)

        def gather_wait(buf, sem):
            pltpu.make_async_copy(
                table_hbm.at[idx_v.at[0]], buf, sem).wait()

        def write_start(c, buf, sem):
            pltpu.async_copy(buf, out_hbm.at[c, pl.ds(col_base, bpw)], sem)

        def write_wait(buf, sem):
            pltpu.make_async_copy(
                buf, out_hbm.at[0, pl.ds(col_base, bpw)], sem).wait()

        for k in range(NB - 1):
            gather_start(k, bufs[k], gs[k])

        def body(i, carry):
            for k in range(NB):
                c = NB * i + k
                gather_wait(bufs[k], gs[k])
                write_start(c, bufs[k], ws[k])
                kk = (k + NB - 1) % NB

                def refill(c=c, kk=kk):
                    write_wait(bufs[kk], ws[kk])
                    gather_start(c + NB - 1, bufs[kk], gs[kk])

                if k == 0:
                    pl.when(i > 0)(refill)
                    pl.when(i == 0)(
                        lambda: gather_start(NB - 1, bufs[NB - 1], gs[NB - 1]))
                else:
                    pl.when(c + NB - 1 < nchunks)(refill)
            return carry

        lax.fori_loop(0, full_groups, body, 0)
        for k in range(rem):
            c = NB * full_groups + k
            gather_wait(bufs[k], gs[k])
            write_start(c, bufs[k], ws[k])
        for k in range(NB):
            write_wait(bufs[k], ws[k])

    return gather_kernel


def kernel(input, weight):
    batch, seq = input.shape
    idx_t = input.astype(jnp.int32).T
    out_t = _make_gather(batch, seq)(idx_t, weight)
    return jnp.transpose(out_t, (1, 0, 2))
